# Initial kernel scaffold; baseline (speedup 1.0000x reference)
import functools

import jax
import jax.numpy as jnp
from jax import lax
from jax.experimental import pallas as pl
from jax.experimental.pallas import tpu as pltpu
from jax.experimental.pallas import tpu_sc as plsc

NC = 2
NS = 16
NW = NC * NS
LANES = 16

_mesh = lambda: plsc.VectorSubcoreMesh(
    core_axis_name="c", subcore_axis_name="s", num_cores=NC, num_subcores=NS)


def _sc_hist(rowp, colp, npad, zero_idx):
    e_pad = rowp.shape[0]
    ept = e_pad // NW

    @functools.partial(
        pl.kernel,
        mesh=_mesh(),
        out_type=jax.ShapeDtypeStruct((NW, npad), jnp.float32),
        scratch_types=[
            pltpu.VMEM((ept,), jnp.int32),
            pltpu.VMEM((ept,), jnp.int32),
            pltpu.VMEM((npad,), jnp.float32),
        ],
    )
    def k(rowp_hbm, colp_hbm, out_hbm, rbuf, cbuf, hist_v):
        cid = lax.axis_index("c")
        sid = lax.axis_index("s")
        wid = cid * NS + sid
        base = wid * ept
        pltpu.sync_copy(rowp_hbm.at[pl.ds(base, ept)], rbuf)
        pltpu.sync_copy(colp_hbm.at[pl.ds(base, ept)], cbuf)

        def zero(i, _):
            hist_v[pl.ds(i * LANES, LANES)] = jnp.zeros((LANES,), jnp.float32)
            return 0
        lax.fori_loop(0, npad // LANES, zero, 0)

        def acc(i, _):
            r = rbuf[pl.ds(i * LANES, LANES)]
            c = cbuf[pl.ds(i * LANES, LANES)]
            val = jnp.where(r != zero_idx, 1.0, 0.0).astype(jnp.float32)
            plsc.addupdate_scatter(hist_v, [c], val)
            return 0
        lax.fori_loop(0, ept // LANES, acc, 0)
        pltpu.sync_copy(hist_v, out_hbm.at[wid])

    return k(rowp, colp)


def _sc_agg(table, rowp, colp, npad):
    e_pad = rowp.shape[0]
    ept = e_pad // NW
    nb = ept // 128
    rows_per_tile = npad // NS

    @functools.partial(
        pl.kernel,
        mesh=_mesh(),
        out_type=jax.ShapeDtypeStruct((NC, npad, 128), jnp.float32),
        scratch_types=[
            pltpu.VMEM((128,), jnp.int32),
            pltpu.VMEM((128,), jnp.int32),
            pltpu.VMEM((128, 128), jnp.float32),
            pltpu.VMEM((64, 128), jnp.float32),
            pltpu.SemaphoreType.DMA,
            pltpu.VMEM_SHARED((npad, 128), jnp.float32),
        ],
    )
    def k(table_hbm, rowp_hbm, colp_hbm, out_hbm, rbuf, cbuf, rows_v, zbuf,
          sem, acc):
        cid = lax.axis_index("c")
        sid = lax.axis_index("s")
        wid = cid * NS + sid

        def zfill(i, _):
            zbuf[i // 8, pl.ds((i % 8) * LANES, LANES)] = (
                jnp.zeros((LANES,), jnp.float32))
            return 0
        lax.fori_loop(0, 64 * 8, zfill, 0)

        def zacc(i, _):
            pltpu.sync_copy(zbuf, acc.at[pl.ds(sid * rows_per_tile + i * 64, 64)])
            return 0
        lax.fori_loop(0, rows_per_tile // 64, zacc, 0)
        plsc.subcore_barrier()

        def body(b, _):
            base = wid * ept + b * 128
            pltpu.sync_copy(rowp_hbm.at[pl.ds(base, 128)], rbuf)
            pltpu.sync_copy(colp_hbm.at[pl.ds(base, 128)], cbuf)
            pltpu.async_copy(table_hbm.at[rbuf], rows_v, sem).wait()
            pltpu.sync_copy(rows_v, acc.at[cbuf], add=True)
            return 0
        lax.fori_loop(0, nb, body, 0)
        plsc.subcore_barrier()
        pltpu.sync_copy(
            acc.at[pl.ds(sid * rows_per_tile, rows_per_tile)],
            out_hbm.at[cid, pl.ds(sid * rows_per_tile, rows_per_tile)])

    return k(table, rowp, colp)


def _tc_prep(x_pad, histp, npad):
    blk = 256
    grid = npad // blk

    def body(x_ref, h_ref, t1_ref, disb_ref):
        deg = jnp.sum(h_ref[...], axis=0) + 1.0
        dis = lax.rsqrt(deg)[:, None]
        t1_ref[...] = x_ref[...] * dis
        disb_ref[...] = jnp.broadcast_to(dis, (blk, 128))

    return pl.pallas_call(
        body,
        grid=(grid,),
        in_specs=[
            pl.BlockSpec((blk, 128), lambda i: (i, 0)),
            pl.BlockSpec((NW, blk), lambda i: (0, i)),
        ],
        out_specs=[
            pl.BlockSpec((blk, 128), lambda i: (i, 0)),
            pl.BlockSpec((blk, 128), lambda i: (i, 0)),
        ],
        out_shape=[
            jax.ShapeDtypeStruct((npad, 128), jnp.float32),
            jax.ShapeDtypeStruct((npad, 128), jnp.float32),
        ],
    )(x_pad, histp)


def _tc_mid(s1p, t1, disb, W1, W2, npad):
    blk = 256
    grid = npad // blk

    def body(s_ref, t1_ref, d_ref, w1_ref, w2_ref, t2_ref):
        s = s_ref[0] + s_ref[1]
        ax = d_ref[...] * (s + t1_ref[...])
        h = jnp.maximum(
            jnp.dot(ax, w1_ref[...], preferred_element_type=jnp.float32), 0.0)
        g = jnp.dot(h, w2_ref[...], preferred_element_type=jnp.float32)
        t2_ref[...] = d_ref[...] * g

    return pl.pallas_call(
        body,
        grid=(grid,),
        in_specs=[
            pl.BlockSpec((NC, blk, 128), lambda i: (0, i, 0)),
            pl.BlockSpec((blk, 128), lambda i: (i, 0)),
            pl.BlockSpec((blk, 128), lambda i: (i, 0)),
            pl.BlockSpec((128, 256), lambda i: (0, 0)),
            pl.BlockSpec((256, 128), lambda i: (0, 0)),
        ],
        out_specs=pl.BlockSpec((blk, 128), lambda i: (i, 0)),
        out_shape=jax.ShapeDtypeStruct((npad, 128), jnp.float32),
    )(s1p, t1, disb, W1, W2)


def _tc_out(s2p, t2, disb, npad):
    blk = 256
    grid = npad // blk

    def body(s_ref, t2_ref, d_ref, o_ref):
        s = s_ref[0] + s_ref[1]
        o = d_ref[...] * (s + t2_ref[...])
        m = jnp.max(o, axis=1, keepdims=True)
        lse = m + jnp.log(jnp.sum(jnp.exp(o - m), axis=1, keepdims=True))
        o_ref[...] = o - lse

    return pl.pallas_call(
        body,
        grid=(grid,),
        in_specs=[
            pl.BlockSpec((NC, blk, 128), lambda i: (0, i, 0)),
            pl.BlockSpec((blk, 128), lambda i: (i, 0)),
            pl.BlockSpec((blk, 128), lambda i: (i, 0)),
        ],
        out_specs=pl.BlockSpec((blk, 128), lambda i: (i, 0)),
        out_shape=jax.ShapeDtypeStruct((npad, 128), jnp.float32),
    )(s2p, t2, disb)


def kernel(x, edge_index, W1, W2):
    N, d_in = x.shape
    E = edge_index.shape[1]
    npad = (N + 256 + 255) // 256 * 256
    e_pad = (E + NW * 128 - 1) // (NW * 128) * (NW * 128)

    row, col = edge_index[0], edge_index[1]
    rowp = jnp.where(row == col, N, row)
    rowp = jnp.concatenate([rowp, jnp.full((e_pad - E,), N, jnp.int32)])
    colp = jnp.concatenate([col, jnp.full((e_pad - E,), N, jnp.int32)])
    x_pad = jnp.pad(x, ((0, npad - N), (0, 0)))

    histp = _sc_hist(rowp, colp, npad, N)
    t1, disb = _tc_prep(x_pad, histp, npad)
    s1p = _sc_agg(t1, rowp, colp, npad)
    t2 = _tc_mid(s1p, t1, disb, W1, W2, npad)
    s2p = _sc_agg(t2, rowp, colp, npad)
    out = _tc_out(s2p, t2, disb, npad)
    return out[:N]


# speedup vs baseline: 12.0377x; 12.0377x over previous
import functools

import jax
import jax.numpy as jnp
from jax import lax
from jax.experimental import pallas as pl
from jax.experimental.pallas import tpu as pltpu
from jax.experimental.pallas import tpu_sc as plsc

NC = 2
NS = 16
NW = NC * NS
LANES = 16

_mesh = lambda: plsc.VectorSubcoreMesh(
    core_axis_name="c", subcore_axis_name="s", num_cores=NC, num_subcores=NS)


def _sc_hist(rowp, colp, npad, zero_idx):
    e_pad = rowp.shape[0]
    ept = e_pad // NW

    @functools.partial(
        pl.kernel,
        mesh=_mesh(),
        out_type=jax.ShapeDtypeStruct((NW, npad), jnp.float32),
        scratch_types=[
            pltpu.VMEM((ept,), jnp.int32),
            pltpu.VMEM((ept,), jnp.int32),
            pltpu.VMEM((npad,), jnp.float32),
        ],
        compiler_params=pltpu.CompilerParams(needs_layout_passes=False),
    )
    def k(rowp_hbm, colp_hbm, out_hbm, rbuf, cbuf, hist_v):
        cid = lax.axis_index("c")
        sid = lax.axis_index("s")
        wid = cid * NS + sid
        base = wid * ept
        pltpu.sync_copy(rowp_hbm.at[pl.ds(base, ept)], rbuf)
        pltpu.sync_copy(colp_hbm.at[pl.ds(base, ept)], cbuf)

        def zero(i, _):
            hist_v[pl.ds(i * LANES, LANES)] = jnp.zeros((LANES,), jnp.float32)
            return 0
        lax.fori_loop(0, npad // LANES, zero, 0)

        def acc(i, _):
            r = rbuf[pl.ds(i * LANES, LANES)]
            c = cbuf[pl.ds(i * LANES, LANES)]
            val = jnp.where(r != zero_idx, 1.0, 0.0).astype(jnp.float32)
            plsc.addupdate_scatter(hist_v, [c], val)
            return 0
        lax.fori_loop(0, ept // LANES, acc, 0)
        pltpu.sync_copy(hist_v, out_hbm.at[wid])

    return k(rowp, colp)


def _sc_agg(table, rowp, colp, npad):
    e_pad = rowp.shape[0]
    ept = e_pad // NW
    nb = ept // 128
    rows_per_tile = npad // NS

    @functools.partial(
        pl.kernel,
        mesh=_mesh(),
        out_type=jax.ShapeDtypeStruct((NC, npad, 128), jnp.float32),
        scratch_types=[
            pltpu.VMEM((128,), jnp.int32),
            pltpu.VMEM((128,), jnp.int32),
            pltpu.VMEM((128, 128), jnp.float32),
            pltpu.VMEM((64, 128), jnp.float32),
            pltpu.SemaphoreType.DMA,
            pltpu.VMEM_SHARED((npad, 128), jnp.float32),
        ],
        compiler_params=pltpu.CompilerParams(needs_layout_passes=False),
    )
    def k(table_hbm, rowp_hbm, colp_hbm, out_hbm, rbuf, cbuf, rows_v, zbuf,
          sem, acc):
        cid = lax.axis_index("c")
        sid = lax.axis_index("s")
        wid = cid * NS + sid

        def zfill(i, _):
            zbuf[i // 8, pl.ds((i % 8) * LANES, LANES)] = (
                jnp.zeros((LANES,), jnp.float32))
            return 0
        lax.fori_loop(0, 64 * 8, zfill, 0)

        def zacc(i, _):
            pltpu.sync_copy(zbuf, acc.at[pl.ds(sid * rows_per_tile + i * 64, 64)])
            return 0
        lax.fori_loop(0, rows_per_tile // 64, zacc, 0)
        plsc.subcore_barrier()

        def body(b, _):
            base = wid * ept + b * 128
            pltpu.sync_copy(rowp_hbm.at[pl.ds(base, 128)], rbuf)
            pltpu.sync_copy(colp_hbm.at[pl.ds(base, 128)], cbuf)
            pltpu.async_copy(table_hbm.at[rbuf], rows_v, sem).wait()
            pltpu.sync_copy(rows_v, acc.at[cbuf], add=True)
            return 0
        lax.fori_loop(0, nb, body, 0)
        plsc.subcore_barrier()
        pltpu.sync_copy(
            acc.at[pl.ds(sid * rows_per_tile, rows_per_tile)],
            out_hbm.at[cid, pl.ds(sid * rows_per_tile, rows_per_tile)])

    return k(table, rowp, colp)


def _tc_prep(x_pad, histp, npad):
    blk = 256
    grid = npad // blk

    def body(x_ref, h_ref, t1_ref, disb_ref):
        deg = jnp.sum(h_ref[...], axis=0) + 1.0
        dis = lax.rsqrt(deg)[:, None]
        t1_ref[...] = x_ref[...] * dis
        disb_ref[...] = jnp.broadcast_to(dis, (blk, 128))

    return pl.pallas_call(
        body,
        grid=(grid,),
        in_specs=[
            pl.BlockSpec((blk, 128), lambda i: (i, 0)),
            pl.BlockSpec((NW, blk), lambda i: (0, i)),
        ],
        out_specs=[
            pl.BlockSpec((blk, 128), lambda i: (i, 0)),
            pl.BlockSpec((blk, 128), lambda i: (i, 0)),
        ],
        out_shape=[
            jax.ShapeDtypeStruct((npad, 128), jnp.float32),
            jax.ShapeDtypeStruct((npad, 128), jnp.float32),
        ],
    )(x_pad, histp)


def _tc_mid(s1p, t1, disb, W1, W2, npad):
    blk = 256
    grid = npad // blk

    def body(s_ref, t1_ref, d_ref, w1_ref, w2_ref, t2_ref):
        s = s_ref[0] + s_ref[1]
        ax = d_ref[...] * (s + t1_ref[...])
        h = jnp.maximum(
            jnp.dot(ax, w1_ref[...], preferred_element_type=jnp.float32), 0.0)
        g = jnp.dot(h, w2_ref[...], preferred_element_type=jnp.float32)
        t2_ref[...] = d_ref[...] * g

    return pl.pallas_call(
        body,
        grid=(grid,),
        in_specs=[
            pl.BlockSpec((NC, blk, 128), lambda i: (0, i, 0)),
            pl.BlockSpec((blk, 128), lambda i: (i, 0)),
            pl.BlockSpec((blk, 128), lambda i: (i, 0)),
            pl.BlockSpec((128, 256), lambda i: (0, 0)),
            pl.BlockSpec((256, 128), lambda i: (0, 0)),
        ],
        out_specs=pl.BlockSpec((blk, 128), lambda i: (i, 0)),
        out_shape=jax.ShapeDtypeStruct((npad, 128), jnp.float32),
    )(s1p, t1, disb, W1, W2)


def _tc_out(s2p, t2, disb, npad):
    blk = 256
    grid = npad // blk

    def body(s_ref, t2_ref, d_ref, o_ref):
        s = s_ref[0] + s_ref[1]
        o = d_ref[...] * (s + t2_ref[...])
        m = jnp.max(o, axis=1, keepdims=True)
        lse = m + jnp.log(jnp.sum(jnp.exp(o - m), axis=1, keepdims=True))
        o_ref[...] = o - lse

    return pl.pallas_call(
        body,
        grid=(grid,),
        in_specs=[
            pl.BlockSpec((NC, blk, 128), lambda i: (0, i, 0)),
            pl.BlockSpec((blk, 128), lambda i: (i, 0)),
            pl.BlockSpec((blk, 128), lambda i: (i, 0)),
        ],
        out_specs=pl.BlockSpec((blk, 128), lambda i: (i, 0)),
        out_shape=jax.ShapeDtypeStruct((npad, 128), jnp.float32),
    )(s2p, t2, disb)


def kernel(x, edge_index, W1, W2):
    N, d_in = x.shape
    E = edge_index.shape[1]
    npad = (N + 1 + 1023) // 1024 * 1024
    e_pad = (E + NW * 128 - 1) // (NW * 128) * (NW * 128)

    row, col = edge_index[0], edge_index[1]
    rowp = jnp.where(row == col, N, row)
    rowp = jnp.concatenate([rowp, jnp.full((e_pad - E,), N, jnp.int32)])
    colp = jnp.concatenate([col, jnp.full((e_pad - E,), N, jnp.int32)])
    x_pad = jnp.pad(x, ((0, npad - N), (0, 0)))

    histp = _sc_hist(rowp, colp, npad, N)
    t1, disb = _tc_prep(x_pad, histp, npad)
    s1p = _sc_agg(t1, rowp, colp, npad)
    t2 = _tc_mid(s1p, t1, disb, W1, W2, npad)
    s2p = _sc_agg(t2, rowp, colp, npad)
    out = _tc_out(s2p, t2, disb, npad)
    return out[:N]
